# Initial kernel scaffold; baseline (speedup 1.0000x reference)
import functools

import jax
import jax.numpy as jnp
from jax import lax
from jax.experimental import pallas as pl
from jax.experimental.pallas import tpu as pltpu
from jax.experimental.pallas import tpu_sc as plsc

N = 10000
E = 320000
D = 128
NC, NS, L = 2, 16, 16
NW = NC * NS
EPW = E // NW
CHUNK = 80
NCHUNK = EPW // CHUNK
RPT = N // NS


def _spmm_sc(x, cols, rows, vals, zeros):
    mesh = plsc.VectorSubcoreMesh(core_axis_name="c", subcore_axis_name="s")

    @functools.partial(
        pl.kernel,
        mesh=mesh,
        out_type=jax.ShapeDtypeStruct((NC, N, D), jnp.float32),
        scratch_types=[
            pltpu.VMEM((CHUNK,), jnp.int32),
            pltpu.VMEM((CHUNK,), jnp.int32),
            pltpu.VMEM((CHUNK,), jnp.float32),
            pltpu.VMEM((CHUNK, D), jnp.float32),
            pltpu.VMEM_SHARED((N, D), jnp.float32),
            pltpu.SemaphoreType.DMA,
        ],
    )
    def k(x_hbm, cols_hbm, rows_hbm, vals_hbm, z_hbm, out_hbm,
          cols_v, rows_v, vals_v, gath_v, agg_sh, sem):
        c = lax.axis_index("c")
        s = lax.axis_index("s")
        wid = s * NC + c
        pltpu.sync_copy(z_hbm.at[pl.ds(s * RPT, RPT)],
                        agg_sh.at[pl.ds(s * RPT, RPT)])
        plsc.subcore_barrier()

        base = wid * EPW

        @pl.loop(0, NCHUNK)
        def _chunk(ch):
            off = pl.multiple_of(base + ch * CHUNK, 8)
            pltpu.sync_copy(cols_hbm.at[pl.ds(off, CHUNK)], cols_v)
            pltpu.sync_copy(rows_hbm.at[pl.ds(off, CHUNK)], rows_v)
            pltpu.sync_copy(vals_hbm.at[pl.ds(off, CHUNK)], vals_v)
            pltpu.async_copy(x_hbm.at[cols_v], gath_v, sem).wait()

            @pl.loop(0, CHUNK)
            def _edge(i):
                val = plsc.load_gather(vals_v, [jnp.full((L,), i, jnp.int32)])
                for d in range(D // L):
                    sl = pl.ds(d * L, L)
                    gath_v[i, sl] = gath_v[i, sl] * val

            pltpu.sync_copy(gath_v, agg_sh.at[rows_v], add=True)

        plsc.subcore_barrier()
        pltpu.sync_copy(agg_sh.at[pl.ds(s * RPT, RPT)],
                        out_hbm.at[c, pl.ds(s * RPT, RPT)])

    return k(x, cols, rows, vals, zeros)


BN = 500


def _fc_tc(p, w, b):
    def body(p_ref, w_ref, b_ref, o_ref):
        a = p_ref[0] + p_ref[1]
        o_ref[...] = lax.dot_general(
            a, w_ref[...], (((1,), (1,)), ((), ())),
            preferred_element_type=jnp.float32) + b_ref[...]

    return pl.pallas_call(
        body,
        grid=(N // BN,),
        in_specs=[pl.BlockSpec((NC, BN, D), lambda i: (0, i, 0)),
                  pl.BlockSpec((D, D), lambda i: (0, 0)),
                  pl.BlockSpec((1, D), lambda i: (0, 0))],
        out_specs=pl.BlockSpec((BN, D), lambda i: (i, 0)),
        out_shape=jax.ShapeDtypeStruct((N, D), jnp.float32),
    )(p, w, b)


def kernel(X, edge_index, edge_vals, W, b):
    rows = edge_index[0].astype(jnp.int32)
    cols = edge_index[1].astype(jnp.int32)
    zeros = jnp.zeros((N, D), jnp.float32)
    p = _spmm_sc(X, cols, rows, edge_vals, zeros)
    return _fc_tc(p, W, jnp.reshape(b, (1, D)))


# speedup vs baseline: 4.4741x; 4.4741x over previous
import functools

import jax
import jax.numpy as jnp
from jax import lax
from jax.experimental import pallas as pl
from jax.experimental.pallas import tpu as pltpu
from jax.experimental.pallas import tpu_sc as plsc

N = 10000
E = 320000
D = 128
NC, NS, L = 2, 16, 16
NW = NC * NS
EPW = E // NW
CHUNK = 80
NCHUNK = EPW // CHUNK
N_PAD = 10240
RPT = N_PAD // NS


def _bcast_lane(v, j):
    idx = jnp.full((L, 1), j, jnp.int32)
    return lax.gather(
        v, idx,
        lax.GatherDimensionNumbers(offset_dims=(), collapsed_slice_dims=(0,),
                                   start_index_map=(0,)),
        (1,), mode=lax.GatherScatterMode.PROMISE_IN_BOUNDS)


def _spmm_sc(x, cols, rows, vals, zeros):
    mesh = plsc.VectorSubcoreMesh(core_axis_name="c", subcore_axis_name="s")

    @functools.partial(
        pl.kernel,
        mesh=mesh,
        out_type=jax.ShapeDtypeStruct((NC, N_PAD, D), jnp.float32),
        scratch_types=[
            pltpu.VMEM((CHUNK,), jnp.int32),
            pltpu.VMEM((CHUNK,), jnp.int32),
            pltpu.VMEM((CHUNK,), jnp.float32),
            pltpu.VMEM((CHUNK, D), jnp.float32),
            pltpu.VMEM_SHARED((N_PAD, D), jnp.float32),
            pltpu.SemaphoreType.DMA,
        ],
    )
    def k(x_hbm, cols_hbm, rows_hbm, vals_hbm, z_hbm, out_hbm,
          cols_v, rows_v, vals_v, gath_v, agg_sh, sem):
        c = lax.axis_index("c")
        s = lax.axis_index("s")
        wid = s * NC + c
        pltpu.sync_copy(z_hbm.at[pl.ds(s * RPT, RPT)],
                        agg_sh.at[pl.ds(s * RPT, RPT)])
        plsc.subcore_barrier()

        base = wid * EPW

        @pl.loop(0, NCHUNK)
        def _chunk(ch):
            off = pl.multiple_of(base + ch * CHUNK, 8)
            pltpu.sync_copy(cols_hbm.at[pl.ds(off, CHUNK)], cols_v)
            pltpu.sync_copy(rows_hbm.at[pl.ds(off, CHUNK)], rows_v)
            pltpu.sync_copy(vals_hbm.at[pl.ds(off, CHUNK)], vals_v)
            pltpu.async_copy(x_hbm.at[cols_v], gath_v, sem).wait()

            @pl.loop(0, CHUNK // L)
            def _grp(g):
                v16 = vals_v[pl.ds(pl.multiple_of(g * L, L), L)]
                for j in range(L):
                    val = _bcast_lane(v16, j)
                    i = g * L + j
                    for d in range(D // L):
                        sl = pl.ds(d * L, L)
                        gath_v[i, sl] = gath_v[i, sl] * val

            pltpu.sync_copy(gath_v, agg_sh.at[rows_v], add=True)

        plsc.subcore_barrier()
        pltpu.sync_copy(agg_sh.at[pl.ds(s * RPT, RPT)],
                        out_hbm.at[c, pl.ds(s * RPT, RPT)])

    return k(x, cols, rows, vals, zeros)


BN = 1000


def _fc_tc(p, w, b):
    def body(p_ref, w_ref, b_ref, o_ref):
        a = p_ref[0] + p_ref[1]
        o_ref[...] = lax.dot_general(
            a, w_ref[...], (((1,), (1,)), ((), ())),
            preferred_element_type=jnp.float32) + b_ref[...]

    return pl.pallas_call(
        body,
        grid=(N // BN,),
        in_specs=[pl.BlockSpec((NC, BN, D), lambda i: (0, i, 0)),

                  pl.BlockSpec((D, D), lambda i: (0, 0)),
                  pl.BlockSpec((1, D), lambda i: (0, 0))],
        out_specs=pl.BlockSpec((BN, D), lambda i: (i, 0)),
        out_shape=jax.ShapeDtypeStruct((N, D), jnp.float32),
    )(p, w, b)


def kernel(X, edge_index, edge_vals, W, b):
    rows = edge_index[0].astype(jnp.int32)
    cols = edge_index[1].astype(jnp.int32)
    zeros = jnp.zeros((N_PAD, D), jnp.float32)
    p = _spmm_sc(X, cols, rows, edge_vals, zeros)
    return _fc_tc(p, W, jnp.reshape(b, (1, D)))


# speedup vs baseline: 10.8554x; 2.4263x over previous
import functools

import jax
import jax.numpy as jnp
from jax import lax
from jax.experimental import pallas as pl
from jax.experimental.pallas import tpu as pltpu
from jax.experimental.pallas import tpu_sc as plsc

N = 10000
E = 320000
D = 128
NC, NS, L = 2, 16, 16
NW = NC * NS
EPW = E // NW
CHUNK = 80
NCHUNK = EPW // CHUNK
N_PAD = 10240
RPT = N_PAD // NS


def _bcast_lane(v, j):
    idx = jnp.full((L, 1), j, jnp.int32)
    return lax.gather(
        v, idx,
        lax.GatherDimensionNumbers(offset_dims=(), collapsed_slice_dims=(0,),
                                   start_index_map=(0,)),
        (1,), mode=lax.GatherScatterMode.PROMISE_IN_BOUNDS)


def _spmm_sc(x, cols, vals, rows, zeros):
    mesh = plsc.VectorSubcoreMesh(core_axis_name="c", subcore_axis_name="s")

    @functools.partial(
        pl.kernel,
        mesh=mesh,
        out_type=jax.ShapeDtypeStruct((NC, N_PAD, D), jnp.float32),
        scratch_types=[
            pltpu.VMEM((NCHUNK, CHUNK), jnp.int32),
            pltpu.VMEM((CHUNK,), jnp.int32),
            pltpu.VMEM((CHUNK,), jnp.int32),
            pltpu.VMEM((CHUNK,), jnp.float32),
            pltpu.VMEM((CHUNK,), jnp.float32),
            pltpu.VMEM((CHUNK, D), jnp.float32),
            pltpu.VMEM((CHUNK, D), jnp.float32),
            pltpu.VMEM_SHARED((N_PAD, D), jnp.float32),
            pltpu.SemaphoreType.DMA,
            pltpu.SemaphoreType.DMA,
            pltpu.SemaphoreType.DMA,
            pltpu.SemaphoreType.DMA,
        ],
    )
    def k(x_hbm, cols_hbm, vals_hbm, rows_hbm, z_hbm, out_hbm,
          rows_v, cb0, cb1, vb0, vb1, gath0, gath1, agg_sh, sg0, sg1, sp0, sp1):
        c = lax.axis_index("c")
        s = lax.axis_index("s")
        wid = s * NC + c
        pltpu.sync_copy(z_hbm.at[pl.ds(s * RPT, RPT)],
                        agg_sh.at[pl.ds(s * RPT, RPT)])
        pltpu.sync_copy(rows_hbm.at[wid], rows_v)
        plsc.subcore_barrier()

        tile_e = wid * EPW

        def stage(ch, cb, vb, sem):
            off = pl.multiple_of(tile_e + ch * CHUNK, 8)
            pltpu.async_copy(cols_hbm.at[pl.ds(off, CHUNK)], cb, sem)
            pltpu.async_copy(vals_hbm.at[pl.ds(off, CHUNK)], vb, sem)

        def wait_stage(cb, vb, sem):
            pltpu.make_async_copy(cols_hbm.at[pl.ds(0, CHUNK)], cb, sem).wait()
            pltpu.make_async_copy(vals_hbm.at[pl.ds(0, CHUNK)], vb, sem).wait()

        def start_gather(cb, buf, sem):
            pltpu.async_copy(x_hbm.at[cb], buf, sem)

        def wait_gather(cb, buf, sem):
            pltpu.make_async_copy(x_hbm.at[cb], buf, sem).wait()

        def scale(vb, buf):
            @pl.loop(0, CHUNK // L)
            def _grp(g):
                off = pl.multiple_of(g * L, 8)
                v16 = vb[pl.ds(off, L)]
                for j in range(L):
                    val = _bcast_lane(v16, j)
                    i = g * L + j
                    for d in range(D // L):
                        sl = pl.ds(d * L, L)
                        buf[i, sl] = buf[i, sl] * val

        def scatter(ch, buf):
            pltpu.sync_copy(buf, agg_sh.at[rows_v.at[ch]], add=True)

        stage(0, cb0, vb0, sp0)
        stage(1, cb1, vb1, sp1)
        wait_stage(cb0, vb0, sp0)
        start_gather(cb0, gath0, sg0)

        @pl.loop(0, (NCHUNK - 1) // 2)
        def _pair(t):
            ch0 = 2 * t
            wait_stage(cb1, vb1, sp1)
            start_gather(cb1, gath1, sg1)
            wait_gather(cb0, gath0, sg0)
            stage(ch0 + 2, cb0, vb0, sp0)
            scale(vb0, gath0)
            scatter(ch0, gath0)
            wait_stage(cb0, vb0, sp0)
            start_gather(cb0, gath0, sg0)
            wait_gather(cb1, gath1, sg1)

            @pl.when(ch0 + 3 < NCHUNK)
            def _():
                stage(ch0 + 3, cb1, vb1, sp1)

            scale(vb1, gath1)
            scatter(ch0 + 1, gath1)

        wait_gather(cb0, gath0, sg0)
        scale(vb0, gath0)
        scatter(NCHUNK - 1, gath0)

        plsc.subcore_barrier()
        pltpu.sync_copy(agg_sh.at[pl.ds(s * RPT, RPT)],
                        out_hbm.at[c, pl.ds(s * RPT, RPT)])

    return k(x, cols, vals, rows, zeros)


BN = 1000


def _fc_tc(p, w, b):
    def body(p_ref, w_ref, b_ref, o_ref):
        a = p_ref[0] + p_ref[1]
        o_ref[...] = lax.dot_general(
            a, w_ref[...], (((1,), (1,)), ((), ())),
            preferred_element_type=jnp.float32) + b_ref[...]

    return pl.pallas_call(
        body,
        grid=(N // BN,),
        in_specs=[pl.BlockSpec((NC, BN, D), lambda i: (0, i, 0)),

                  pl.BlockSpec((D, D), lambda i: (0, 0)),
                  pl.BlockSpec((1, D), lambda i: (0, 0))],
        out_specs=pl.BlockSpec((BN, D), lambda i: (i, 0)),
        out_shape=jax.ShapeDtypeStruct((N, D), jnp.float32),
    )(p, w, b)


def kernel(X, edge_index, edge_vals, W, b):
    rows = edge_index[0].astype(jnp.int32).reshape(NW, NCHUNK, CHUNK)
    cols = edge_index[1].astype(jnp.int32)
    zeros = jnp.zeros((N_PAD, D), jnp.float32)
    p = _spmm_sc(X, cols, edge_vals, rows, zeros)
    return _fc_tc(p, W, jnp.reshape(b, (1, D)))
